# baseline (speedup 1.0000x reference)
import functools

import jax
import jax.numpy as jnp
from jax import lax
from jax.experimental import pallas as pl
from jax.experimental.pallas import tpu as pltpu
from jax.experimental.pallas import tpu_sc as plsc

NC, NS, L = 2, 16, 16
NW = NC * NS
ROW1 = 144
ROW2 = 48
CH = 80


def _iota16():
    return lax.iota(jnp.int32, 16)


def _lane_gather(v, idx):
    return lax.gather(
        v, idx[:, None],
        dimension_numbers=lax.GatherDimensionNumbers(
            offset_dims=(), collapsed_slice_dims=(0,), start_index_map=(0,)),
        slice_sizes=(1,), mode=lax.GatherScatterMode.PROMISE_IN_BOUNDS)


def _splat(v, j):
    return _lane_gather(v, jnp.full((16,), j, dtype=jnp.int32))


ACC1 = 144
ACC2 = 48


def _make_edge_pass(SRCW, DW, ACCW, CH, compute_chunk):
    def run(srctab, dsttab, srcidx3, dstidx3, zeros):
        N = srctab.shape[0]
        steps = srcidx3.shape[1]
        rps = N // NS
        mesh = plsc.VectorSubcoreMesh(core_axis_name="c", subcore_axis_name="s",
                                      num_cores=NC, num_subcores=NS)

        @functools.partial(
            pl.kernel,
            out_type=jax.ShapeDtypeStruct((NC * N, ACCW), jnp.float32),
            mesh=mesh,
            scratch_types=[
                pltpu.VMEM((4, CH), jnp.int32),
                pltpu.VMEM((4, CH), jnp.int32),
                pltpu.VMEM((2, CH, SRCW), jnp.float32),
                pltpu.VMEM((2, CH, DW), jnp.float32),
                pltpu.VMEM((2, CH, ACCW), jnp.float32),
                pltpu.VMEM_SHARED((N, ACCW), jnp.float32),
                pltpu.SemaphoreType.DMA((4,)),
                pltpu.SemaphoreType.DMA((2,)),
                pltpu.SemaphoreType.DMA((2,)),
                pltpu.SemaphoreType.DMA((2,)),
            ],
            compiler_params=pltpu.CompilerParams(
                use_tc_tiling_on_sc=False, needs_layout_passes=False),
        )
        def k(srctab_hbm, dsttab_hbm, sidx_hbm, didx_hbm, zeros_hbm, out_hbm,
              sidx, didx, srows, drows, contrib, accum, isem, gs, gd, ssem):
            c = lax.axis_index("c")
            s = lax.axis_index("s")
            wid = s * NC + c
            r0 = s * rps

            pltpu.sync_copy(zeros_hbm, accum.at[pl.ds(r0, rps)])

            def issue_idx(t):
                slot = jnp.bitwise_and(t, 3)
                pltpu.async_copy(sidx_hbm.at[wid, t], sidx.at[slot],
                                 isem.at[slot])
                pltpu.async_copy(didx_hbm.at[wid, t], didx.at[slot],
                                 isem.at[slot])

            def wait_idx(t):
                slot = jnp.bitwise_and(t, 3)
                pltpu.make_async_copy(sidx_hbm.at[wid, t], sidx.at[slot],
                                      isem.at[slot]).wait()
                pltpu.make_async_copy(didx_hbm.at[wid, t], didx.at[slot],
                                      isem.at[slot]).wait()

            def issue_g(t, p):
                slot = jnp.bitwise_and(t, 3)
                pltpu.async_copy(srctab_hbm.at[sidx.at[slot]], srows.at[p],
                                 gs.at[p])
                pltpu.async_copy(dsttab_hbm.at[didx.at[slot]], drows.at[p],
                                 gd.at[p])

            def wait_g(t, p):
                slot = jnp.bitwise_and(t, 3)
                pltpu.make_async_copy(srctab_hbm.at[sidx.at[slot]],
                                      srows.at[p], gs.at[p]).wait()
                pltpu.make_async_copy(dsttab_hbm.at[didx.at[slot]],
                                      drows.at[p], gd.at[p]).wait()

            plsc.subcore_barrier()
            issue_idx(0)
            issue_idx(1)
            wait_idx(0)
            issue_g(0, 0)

            def drain_scatter(t):
                slot = jnp.bitwise_and(t, 3)
                p = jnp.bitwise_and(t, 1)
                pltpu.make_async_copy(contrib.at[p],
                                      accum.at[didx.at[slot]],
                                      ssem.at[p]).wait()

            def step(t, _):
                p = jnp.bitwise_and(t, 1)

                @pl.when(t + 2 < steps)
                def _():
                    issue_idx(t + 2)

                @pl.when(t + 1 < steps)
                def _():
                    wait_idx(t + 1)
                    issue_g(t + 1, 1 - p)
                wait_g(t, p)

                @pl.when(t >= 2)
                def _():
                    drain_scatter(t - 2)

                @pl.when(p == 0)
                def _():
                    compute_chunk(srows.at[0], drows.at[0], contrib.at[0])

                @pl.when(p == 1)
                def _():
                    compute_chunk(srows.at[1], drows.at[1], contrib.at[1])
                slot = jnp.bitwise_and(t, 3)
                pltpu.async_copy(contrib.at[p], accum.at[didx.at[slot]],
                                 ssem.at[p], add=True)
                return 0
            lax.fori_loop(0, steps, step, 0)
            drain_scatter(steps - 2)
            drain_scatter(steps - 1)
            plsc.subcore_barrier()

            pltpu.sync_copy(accum.at[pl.ds(r0, rps)],
                            out_hbm.at[pl.ds(c * N + r0, rps)])

        return k(srctab, dsttab, srcidx3, dstidx3, zeros)

    return run


def _compute1(CH):
    def f(srows, drows, contrib):
        cvec = jnp.where(_iota16() < 8, 1.0, 0.2).astype(jnp.float32)
        swap = jnp.bitwise_and(_iota16() + 8, 15)

        @plsc.parallel_loop(0, CH, 1, unroll=4)
        def edge(i):
            el16 = srows[i, pl.ds(0, 16)]
            ab = drows[i, :]
            q = el16 * cvec + ab
            w16 = jnp.exp(jnp.maximum(q, _lane_gather(q, swap)))
            contrib[i, pl.ds(0, 16)] = w16
            for hh in range(8):
                hv = srows[i, pl.ds(16 + 16 * hh, 16)]
                contrib[i, pl.ds(16 + 16 * hh, 16)] = hv * _splat(w16, hh)
    return f


def _compute2(CH):
    def f(srows, drows, contrib):
        iota = _iota16()
        zeros_i = jnp.zeros((16,), jnp.int32)
        ones_i = jnp.ones((16,), jnp.int32)

        @plsc.parallel_loop(0, CH // 16, 1)
        def group(g):
            evec = g * 16 + iota
            el16 = plsc.load_gather(srows, [evec, zeros_i])
            a16 = plsc.load_gather(drows, [evec, zeros_i])
            b16 = plsc.load_gather(drows, [evec, ones_i])
            w16 = jnp.exp(jnp.maximum(el16 + a16, 0.2 * el16 + b16))
            for j in range(16):
                e = g * 16 + j
                wsp = _splat(w16, j)
                row0 = srows[e, pl.ds(0, 16)]
                row0 = jnp.where(iota == 0, 1.0, row0)
                contrib[e, pl.ds(0, 16)] = row0 * wsp
                contrib[e, pl.ds(16, 16)] = srows[e, pl.ds(16, 16)] * wsp
                contrib[e, pl.ds(32, 16)] = srows[e, pl.ds(32, 16)] * wsp
    return f


CH1, CH2 = 40, 80
_edge_pass1_run = _make_edge_pass(ROW1, 16, ACC1, CH1, _compute1(CH1))
_edge_pass2_run = _make_edge_pass(ROW2, 8, ACC2, CH2, _compute2(CH2))


def _tc_prep1(x, W1, ALR, interpret=False):
    N = x.shape[0]
    BN = 1000

    def body(x_ref, w_ref, alr_ref, src_ref, dst_ref, mx_ref):
        ph = pl.program_id(0)
        i = pl.program_id(1)
        xb = x_ref[...]
        hb = jnp.dot(xb, w_ref[...], preferred_element_type=jnp.float32)
        elr = jnp.dot(hb, alr_ref[...], preferred_element_type=jnp.float32)
        el = elr[:, 0:8]
        er = elr[:, 8:16]
        src_ref[...] = jnp.concatenate([el, el, hb], axis=1)
        bmax = jnp.max(el, axis=0, keepdims=True)

        @pl.when(jnp.logical_and(ph == 0, i == 0))
        def _():
            mx_ref[...] = bmax

        @pl.when(jnp.logical_and(ph == 0, i > 0))
        def _():
            mx_ref[...] = jnp.maximum(mx_ref[...], bmax)

        @pl.when(ph == 1)
        def _():
            t = er + mx_ref[...]
            m = jnp.where(t > 0, t, 0.2 * t)
            dst_ref[...] = jnp.concatenate([er - m, 0.2 * er - m], axis=1)

    return pl.pallas_call(
        body,
        grid=(2, N // BN),
        in_specs=[pl.BlockSpec((BN, 128), lambda p, i: (i, 0)),
                  pl.BlockSpec((128, 128), lambda p, i: (0, 0)),
                  pl.BlockSpec((128, 16), lambda p, i: (0, 0))],
        out_specs=[pl.BlockSpec((BN, ROW1), lambda p, i: (i, 0)),
                   pl.BlockSpec((BN, 16), lambda p, i: (i, 0)),
                   pl.BlockSpec((1, 8), lambda p, i: (0, 0))],
        out_shape=[jax.ShapeDtypeStruct((N, ROW1), jnp.float32),
                   jax.ShapeDtypeStruct((N, 16), jnp.float32),
                   jax.ShapeDtypeStruct((1, 8), jnp.float32)],
        interpret=interpret,
    )(x, W1, ALR)


def _tc_prep2(p0, p1, b1, W2, al2, ar2, resW2, b2, interpret=False):
    N = p0.shape[0]
    BN = 1000

    def body(p0_ref, p1_ref, b1_ref, w2_ref, al2_ref, ar2_ref, rw_ref, b2_ref,
             src_ref, res_ref, dst_ref, mx_ref):
        ph = pl.program_id(0)
        i = pl.program_id(1)
        acc = p0_ref[...] + p1_ref[...]
        num = acc[:, 16:ACC1]
        parts = []
        for hh in range(8):
            dh = acc[:, hh:hh + 1]
            parts.append(num[:, 16 * hh:16 * hh + 16] / (dh + 1e-16))
        rst = jnp.concatenate(parts, axis=1) + b1_ref[...]
        h2 = jnp.where(rst > 0, rst, jnp.exp(rst) - 1.0)
        h2w = jnp.dot(h2, w2_ref[...], preferred_element_type=jnp.float32)
        el2 = jnp.dot(h2w, al2_ref[...], preferred_element_type=jnp.float32)
        er2 = jnp.dot(h2w, ar2_ref[...], preferred_element_type=jnp.float32)
        res = jnp.dot(h2, rw_ref[...], preferred_element_type=jnp.float32)
        res_ref[...] = res + b2_ref[...]
        src_ref[...] = jnp.concatenate(
            [el2, h2w, jnp.zeros((h2w.shape[0], 7), jnp.float32)], axis=1)
        bmax = jnp.max(el2, axis=0, keepdims=True)

        @pl.when(jnp.logical_and(ph == 0, i == 0))
        def _():
            mx_ref[...] = bmax

        @pl.when(jnp.logical_and(ph == 0, i > 0))
        def _():
            mx_ref[...] = jnp.maximum(mx_ref[...], bmax)

        @pl.when(ph == 1)
        def _():
            t = er2 + mx_ref[...]
            m = jnp.where(t > 0, t, 0.2 * t)
            dst_ref[...] = jnp.concatenate(
                [er2 - m, 0.2 * er2 - m,
                 jnp.zeros((er2.shape[0], 6), jnp.float32)], axis=1)

    return pl.pallas_call(
        body,
        grid=(2, N // BN),
        in_specs=[pl.BlockSpec((BN, ACC1), lambda p, i: (i, 0)),
                  pl.BlockSpec((BN, ACC1), lambda p, i: (i, 0)),
                  pl.BlockSpec((1, 128), lambda p, i: (0, 0)),
                  pl.BlockSpec((128, 40), lambda p, i: (0, 0)),
                  pl.BlockSpec((40, 1), lambda p, i: (0, 0)),
                  pl.BlockSpec((40, 1), lambda p, i: (0, 0)),
                  pl.BlockSpec((128, 40), lambda p, i: (0, 0)),
                  pl.BlockSpec((1, 40), lambda p, i: (0, 0))],
        out_specs=[pl.BlockSpec((BN, ROW2), lambda p, i: (i, 0)),
                   pl.BlockSpec((BN, 40), lambda p, i: (i, 0)),
                   pl.BlockSpec((BN, 8), lambda p, i: (i, 0)),
                   pl.BlockSpec((1, 1), lambda p, i: (0, 0))],
        out_shape=[jax.ShapeDtypeStruct((N, ROW2), jnp.float32),
                   jax.ShapeDtypeStruct((N, 40), jnp.float32),
                   jax.ShapeDtypeStruct((N, 8), jnp.float32),
                   jax.ShapeDtypeStruct((1, 1), jnp.float32)],
        interpret=interpret,
    )(p0, p1, b1, W2, al2, ar2, resW2, b2)


def _tc_final(q0, q1, res, interpret=False):
    N = q0.shape[0]
    BN = 1000

    def body(q0_ref, q1_ref, res_ref, out_ref):
        acc = q0_ref[...] + q1_ref[...]
        out_ref[...] = acc[:, 1:41] / (acc[:, 0:1] + 1e-16) + res_ref[...]

    return pl.pallas_call(
        body,
        grid=(N // BN,),
        in_specs=[pl.BlockSpec((BN, ROW2), lambda i: (i, 0)),
                  pl.BlockSpec((BN, ROW2), lambda i: (i, 0)),
                  pl.BlockSpec((BN, 40), lambda i: (i, 0))],
        out_specs=pl.BlockSpec((BN, 40), lambda i: (i, 0)),
        out_shape=jax.ShapeDtypeStruct((N, 40), jnp.float32),
        interpret=interpret,
    )(q0, q1, res)


def kernel(x, edge_index, W1, al1, ar1, b1, W2, al2, ar2, b2, resW2):
    N = x.shape[0]
    src = edge_index[0]
    dst = edge_index[1]

    alr_l = (jnp.eye(8, dtype=jnp.float32)[:, None, :] * al1[:, :, None]
             ).reshape(128, 8)
    alr_r = (jnp.eye(8, dtype=jnp.float32)[:, None, :] * ar1[:, :, None]
             ).reshape(128, 8)
    ALR = jnp.concatenate([alr_l, alr_r], axis=1)

    E = src.shape[0]
    EW = E // NW
    src1 = src.reshape(NW, EW // CH1, CH1)
    dst1 = dst.reshape(NW, EW // CH1, CH1)
    src2 = src.reshape(NW, EW // CH2, CH2)
    dst2 = dst.reshape(NW, EW // CH2, CH2)
    zeros1 = jnp.zeros((N // NS, ACC1), jnp.float32)
    zeros2 = jnp.zeros((N // NS, ACC2), jnp.float32)

    srctab1, dsttab1, _ = _tc_prep1(x, W1, ALR)
    part1 = _edge_pass1_run(srctab1, dsttab1, src1, dst1, zeros1)
    p0, p1 = part1[:N], part1[N:]

    srctab2, res2, dsttab2, _ = _tc_prep2(
        p0, p1, b1.reshape(1, 128), W2, al2.reshape(40, 1), ar2.reshape(40, 1),
        resW2, b2.reshape(1, 40))
    part2 = _edge_pass2_run(srctab2, dsttab2, src2, dst2, zeros2)

    return _tc_final(part2[:N], part2[N:], res2)


# speedup vs baseline: 1.0721x; 1.0721x over previous
import functools

import jax
import jax.numpy as jnp
from jax import lax
from jax.experimental import pallas as pl
from jax.experimental.pallas import tpu as pltpu
from jax.experimental.pallas import tpu_sc as plsc

NC, NS, L = 2, 16, 16
NW = NC * NS
ROW1 = 144
ROW2 = 48
CH = 80


def _iota16():
    return lax.iota(jnp.int32, 16)


def _lane_gather(v, idx):
    return lax.gather(
        v, idx[:, None],
        dimension_numbers=lax.GatherDimensionNumbers(
            offset_dims=(), collapsed_slice_dims=(0,), start_index_map=(0,)),
        slice_sizes=(1,), mode=lax.GatherScatterMode.PROMISE_IN_BOUNDS)


def _splat(v, j):
    return _lane_gather(v, jnp.full((16,), j, dtype=jnp.int32))


ACC1 = 144
ACC2 = 48


def _make_edge_pass(SRCW, DW, ACCW, CH, compute_chunk, async_scatter=True):
    def run(srctab, dsttab, srcidx3, dstidx3, zeros):
        N = srctab.shape[0]
        steps = srcidx3.shape[1]
        rps = N // NS
        mesh = plsc.VectorSubcoreMesh(core_axis_name="c", subcore_axis_name="s",
                                      num_cores=NC, num_subcores=NS)

        @functools.partial(
            pl.kernel,
            out_type=jax.ShapeDtypeStruct((NC * N, ACCW), jnp.float32),
            mesh=mesh,
            scratch_types=[
                pltpu.VMEM((4, CH), jnp.int32),
                pltpu.VMEM((4, CH), jnp.int32),
                pltpu.VMEM((2, CH, SRCW), jnp.float32),
                pltpu.VMEM((2, CH, DW), jnp.float32),
                pltpu.VMEM((2 if async_scatter else 1, CH, ACCW), jnp.float32),
                pltpu.VMEM_SHARED((N, ACCW), jnp.float32),
                pltpu.SemaphoreType.DMA((4,)),
                pltpu.SemaphoreType.DMA((2,)),
                pltpu.SemaphoreType.DMA((2,)),
                pltpu.SemaphoreType.DMA((2,)),
            ],
            compiler_params=pltpu.CompilerParams(
                use_tc_tiling_on_sc=False, needs_layout_passes=False),
        )
        def k(srctab_hbm, dsttab_hbm, sidx_hbm, didx_hbm, zeros_hbm, out_hbm,
              sidx, didx, srows, drows, contrib, accum, isem, gs, gd, ssem):
            c = lax.axis_index("c")
            s = lax.axis_index("s")
            wid = s * NC + c
            r0 = s * rps

            pltpu.sync_copy(zeros_hbm, accum.at[pl.ds(r0, rps)])

            def issue_idx(t):
                slot = jnp.bitwise_and(t, 3)
                pltpu.async_copy(sidx_hbm.at[wid, t], sidx.at[slot],
                                 isem.at[slot])
                pltpu.async_copy(didx_hbm.at[wid, t], didx.at[slot],
                                 isem.at[slot])

            def wait_idx(t):
                slot = jnp.bitwise_and(t, 3)
                pltpu.make_async_copy(sidx_hbm.at[wid, t], sidx.at[slot],
                                      isem.at[slot]).wait()
                pltpu.make_async_copy(didx_hbm.at[wid, t], didx.at[slot],
                                      isem.at[slot]).wait()

            def issue_g(t, p):
                slot = jnp.bitwise_and(t, 3)
                pltpu.async_copy(srctab_hbm.at[sidx.at[slot]], srows.at[p],
                                 gs.at[p])
                pltpu.async_copy(dsttab_hbm.at[didx.at[slot]], drows.at[p],
                                 gd.at[p])

            def wait_g(t, p):
                slot = jnp.bitwise_and(t, 3)
                pltpu.make_async_copy(srctab_hbm.at[sidx.at[slot]],
                                      srows.at[p], gs.at[p]).wait()
                pltpu.make_async_copy(dsttab_hbm.at[didx.at[slot]],
                                      drows.at[p], gd.at[p]).wait()

            plsc.subcore_barrier()
            issue_idx(0)
            issue_idx(1)
            wait_idx(0)
            issue_g(0, 0)

            def drain_scatter(t):
                slot = jnp.bitwise_and(t, 3)
                p = jnp.bitwise_and(t, 1)
                pltpu.make_async_copy(contrib.at[p],
                                      accum.at[didx.at[slot]],
                                      ssem.at[p]).wait()

            def step(t, _):
                p = jnp.bitwise_and(t, 1)

                @pl.when(t + 2 < steps)
                def _():
                    issue_idx(t + 2)

                @pl.when(t + 1 < steps)
                def _():
                    wait_idx(t + 1)
                    issue_g(t + 1, 1 - p)
                wait_g(t, p)
                slot = jnp.bitwise_and(t, 3)
                if async_scatter:
                    @pl.when(t >= 2)
                    def _():
                        drain_scatter(t - 2)

                    @pl.when(p == 0)
                    def _():
                        compute_chunk(srows.at[0], drows.at[0], contrib.at[0])

                    @pl.when(p == 1)
                    def _():
                        compute_chunk(srows.at[1], drows.at[1], contrib.at[1])
                    pltpu.async_copy(contrib.at[p], accum.at[didx.at[slot]],
                                     ssem.at[p], add=True)
                else:
                    @pl.when(p == 0)
                    def _():
                        compute_chunk(srows.at[0], drows.at[0], contrib.at[0])

                    @pl.when(p == 1)
                    def _():
                        compute_chunk(srows.at[1], drows.at[1], contrib.at[0])
                    pltpu.sync_copy(contrib.at[0], accum.at[didx.at[slot]],
                                    add=True)
                return 0
            lax.fori_loop(0, steps, step, 0)
            if async_scatter:
                drain_scatter(steps - 2)
                drain_scatter(steps - 1)
            plsc.subcore_barrier()

            pltpu.sync_copy(accum.at[pl.ds(r0, rps)],
                            out_hbm.at[pl.ds(c * N + r0, rps)])

        return k(srctab, dsttab, srcidx3, dstidx3, zeros)

    return run


def _compute1(CH):
    def f(srows, drows, contrib):
        cvec = jnp.where(_iota16() < 8, 1.0, 0.2).astype(jnp.float32)
        swap = jnp.bitwise_and(_iota16() + 8, 15)

        @plsc.parallel_loop(0, CH, 1, unroll=4)
        def edge(i):
            el16 = srows[i, pl.ds(0, 16)]
            ab = drows[i, :]
            q = el16 * cvec + ab
            w16 = jnp.exp(jnp.maximum(q, _lane_gather(q, swap)))
            contrib[i, pl.ds(0, 16)] = w16
            for hh in range(8):
                hv = srows[i, pl.ds(16 + 16 * hh, 16)]
                contrib[i, pl.ds(16 + 16 * hh, 16)] = hv * _splat(w16, hh)
    return f


def _compute2(CH):
    def f(srows, drows, contrib):
        iota = _iota16()
        zeros_i = jnp.zeros((16,), jnp.int32)
        ones_i = jnp.ones((16,), jnp.int32)

        @plsc.parallel_loop(0, CH // 16, 1)
        def group(g):
            evec = g * 16 + iota
            el16 = plsc.load_gather(srows, [evec, zeros_i])
            a16 = plsc.load_gather(drows, [evec, zeros_i])
            b16 = plsc.load_gather(drows, [evec, ones_i])
            w16 = jnp.exp(jnp.maximum(el16 + a16, 0.2 * el16 + b16))
            for j in range(16):
                e = g * 16 + j
                wsp = _splat(w16, j)
                row0 = srows[e, pl.ds(0, 16)]
                row0 = jnp.where(iota == 0, 1.0, row0)
                contrib[e, pl.ds(0, 16)] = row0 * wsp
                contrib[e, pl.ds(16, 16)] = srows[e, pl.ds(16, 16)] * wsp
                contrib[e, pl.ds(32, 16)] = srows[e, pl.ds(32, 16)] * wsp
    return f


CH1, CH2 = 80, 80
_edge_pass1_run = _make_edge_pass(ROW1, 16, ACC1, CH1, _compute1(CH1),
                                  async_scatter=False)
_edge_pass2_run = _make_edge_pass(ROW2, 8, ACC2, CH2, _compute2(CH2))


def _tc_prep1(x, W1, ALR, interpret=False):
    N = x.shape[0]
    BN = 1000

    def body(x_ref, w_ref, alr_ref, src_ref, elr_ref, mx_ref):
        i = pl.program_id(0)
        xb = x_ref[...]
        hb = jnp.dot(xb, w_ref[...], preferred_element_type=jnp.float32)
        elr = jnp.dot(hb, alr_ref[...], preferred_element_type=jnp.float32)
        el = elr[:, 0:8]
        src_ref[...] = jnp.concatenate([el, el, hb], axis=1)
        elr_ref[...] = elr
        bmax = jnp.max(el, axis=0, keepdims=True)

        @pl.when(i == 0)
        def _():
            mx_ref[...] = bmax

        @pl.when(i > 0)
        def _():
            mx_ref[...] = jnp.maximum(mx_ref[...], bmax)

    return pl.pallas_call(
        body,
        grid=(N // BN,),
        in_specs=[pl.BlockSpec((BN, 128), lambda i: (i, 0)),
                  pl.BlockSpec((128, 128), lambda i: (0, 0)),
                  pl.BlockSpec((128, 16), lambda i: (0, 0))],
        out_specs=[pl.BlockSpec((BN, ROW1), lambda i: (i, 0)),
                   pl.BlockSpec((BN, 16), lambda i: (i, 0)),
                   pl.BlockSpec((1, 8), lambda i: (0, 0))],
        out_shape=[jax.ShapeDtypeStruct((N, ROW1), jnp.float32),
                   jax.ShapeDtypeStruct((N, 16), jnp.float32),
                   jax.ShapeDtypeStruct((1, 8), jnp.float32)],
        interpret=interpret,
    )(x, W1, ALR)


def _tc_dsttab1(elr, maxel, interpret=False):
    N = elr.shape[0]
    BN = 1000

    def body(elr_ref, mx_ref, out_ref):
        er = elr_ref[...][:, 8:16]
        t = er + mx_ref[...]
        m = jnp.where(t > 0, t, 0.2 * t)
        out_ref[...] = jnp.concatenate([er - m, 0.2 * er - m], axis=1)

    return pl.pallas_call(
        body,
        grid=(N // BN,),
        in_specs=[pl.BlockSpec((BN, 16), lambda i: (i, 0)),
                  pl.BlockSpec((1, 8), lambda i: (0, 0))],
        out_specs=pl.BlockSpec((BN, 16), lambda i: (i, 0)),
        out_shape=jax.ShapeDtypeStruct((N, 16), jnp.float32),
        interpret=interpret,
    )(elr, maxel)


def _tc_prep2(p0, p1, b1, W2, al2, ar2, resW2, b2, interpret=False):
    N = p0.shape[0]
    BN = 1000

    def body(p0_ref, p1_ref, b1_ref, w2_ref, al2_ref, ar2_ref, rw_ref, b2_ref,
             src_ref, elr_ref, res_ref, mx_ref):
        i = pl.program_id(0)
        acc = p0_ref[...] + p1_ref[...]
        num = acc[:, 16:ACC1]
        parts = []
        for hh in range(8):
            dh = acc[:, hh:hh + 1]
            parts.append(num[:, 16 * hh:16 * hh + 16] / (dh + 1e-16))
        rst = jnp.concatenate(parts, axis=1) + b1_ref[...]
        h2 = jnp.where(rst > 0, rst, jnp.exp(rst) - 1.0)
        h2w = jnp.dot(h2, w2_ref[...], preferred_element_type=jnp.float32)
        el2 = jnp.dot(h2w, al2_ref[...], preferred_element_type=jnp.float32)
        er2 = jnp.dot(h2w, ar2_ref[...], preferred_element_type=jnp.float32)
        res = jnp.dot(h2, rw_ref[...], preferred_element_type=jnp.float32)
        res_ref[...] = res + b2_ref[...]
        src_ref[...] = jnp.concatenate(
            [el2, h2w, jnp.zeros((h2w.shape[0], 7), jnp.float32)], axis=1)
        elr_ref[...] = jnp.concatenate(
            [el2, er2, jnp.zeros((h2w.shape[0], 14), jnp.float32)], axis=1)
        bmax = jnp.max(el2, axis=0, keepdims=True)

        @pl.when(i == 0)
        def _():
            mx_ref[...] = bmax

        @pl.when(i > 0)
        def _():
            mx_ref[...] = jnp.maximum(mx_ref[...], bmax)

    return pl.pallas_call(
        body,
        grid=(N // BN,),
        in_specs=[pl.BlockSpec((BN, ACC1), lambda i: (i, 0)),
                  pl.BlockSpec((BN, ACC1), lambda i: (i, 0)),
                  pl.BlockSpec((1, 128), lambda i: (0, 0)),
                  pl.BlockSpec((128, 40), lambda i: (0, 0)),
                  pl.BlockSpec((40, 1), lambda i: (0, 0)),
                  pl.BlockSpec((40, 1), lambda i: (0, 0)),
                  pl.BlockSpec((128, 40), lambda i: (0, 0)),
                  pl.BlockSpec((1, 40), lambda i: (0, 0))],
        out_specs=[pl.BlockSpec((BN, ROW2), lambda i: (i, 0)),
                   pl.BlockSpec((BN, 16), lambda i: (i, 0)),
                   pl.BlockSpec((BN, 40), lambda i: (i, 0)),
                   pl.BlockSpec((1, 1), lambda i: (0, 0))],
        out_shape=[jax.ShapeDtypeStruct((N, ROW2), jnp.float32),
                   jax.ShapeDtypeStruct((N, 16), jnp.float32),
                   jax.ShapeDtypeStruct((N, 40), jnp.float32),
                   jax.ShapeDtypeStruct((1, 1), jnp.float32)],
        interpret=interpret,
    )(p0, p1, b1, W2, al2, ar2, resW2, b2)


def _tc_dsttab2(elr2, maxel2, interpret=False):
    N = elr2.shape[0]
    BN = 1000

    def body(elr_ref, mx_ref, out_ref):
        e = elr_ref[...]
        er2 = e[:, 1:2]
        t = er2 + mx_ref[...]
        m = jnp.where(t > 0, t, 0.2 * t)
        out_ref[...] = jnp.concatenate(
            [er2 - m, 0.2 * er2 - m,
             jnp.zeros((e.shape[0], 6), jnp.float32)], axis=1)

    return pl.pallas_call(
        body,
        grid=(N // BN,),
        in_specs=[pl.BlockSpec((BN, 16), lambda i: (i, 0)),
                  pl.BlockSpec((1, 1), lambda i: (0, 0))],
        out_specs=pl.BlockSpec((BN, 8), lambda i: (i, 0)),
        out_shape=jax.ShapeDtypeStruct((N, 8), jnp.float32),
        interpret=interpret,
    )(elr2, maxel2)


def _tc_final(q0, q1, res, interpret=False):
    N = q0.shape[0]
    BN = 1000

    def body(q0_ref, q1_ref, res_ref, out_ref):
        acc = q0_ref[...] + q1_ref[...]
        out_ref[...] = acc[:, 1:41] / (acc[:, 0:1] + 1e-16) + res_ref[...]

    return pl.pallas_call(
        body,
        grid=(N // BN,),
        in_specs=[pl.BlockSpec((BN, ROW2), lambda i: (i, 0)),
                  pl.BlockSpec((BN, ROW2), lambda i: (i, 0)),
                  pl.BlockSpec((BN, 40), lambda i: (i, 0))],
        out_specs=pl.BlockSpec((BN, 40), lambda i: (i, 0)),
        out_shape=jax.ShapeDtypeStruct((N, 40), jnp.float32),
        interpret=interpret,
    )(q0, q1, res)


def kernel(x, edge_index, W1, al1, ar1, b1, W2, al2, ar2, b2, resW2):
    N = x.shape[0]
    src = edge_index[0]
    dst = edge_index[1]

    alr_l = (jnp.eye(8, dtype=jnp.float32)[:, None, :] * al1[:, :, None]
             ).reshape(128, 8)
    alr_r = (jnp.eye(8, dtype=jnp.float32)[:, None, :] * ar1[:, :, None]
             ).reshape(128, 8)
    ALR = jnp.concatenate([alr_l, alr_r], axis=1)

    E = src.shape[0]
    EW = E // NW
    src1 = src.reshape(NW, EW // CH1, CH1)
    dst1 = dst.reshape(NW, EW // CH1, CH1)
    src2 = src.reshape(NW, EW // CH2, CH2)
    dst2 = dst.reshape(NW, EW // CH2, CH2)
    zeros1 = jnp.zeros((N // NS, ACC1), jnp.float32)
    zeros2 = jnp.zeros((N // NS, ACC2), jnp.float32)

    srctab1, elr1, maxel1 = _tc_prep1(x, W1, ALR)
    dsttab1 = _tc_dsttab1(elr1, maxel1)
    part1 = _edge_pass1_run(srctab1, dsttab1, src1, dst1, zeros1)
    p0, p1 = part1[:N], part1[N:]

    srctab2, elr2, res2, maxel2 = _tc_prep2(
        p0, p1, b1.reshape(1, 128), W2, al2.reshape(40, 1), ar2.reshape(40, 1),
        resW2, b2.reshape(1, 40))
    dsttab2 = _tc_dsttab2(elr2, maxel2)
    part2 = _edge_pass2_run(srctab2, dsttab2, src2, dst2, zeros2)

    return _tc_final(part2[:N], part2[N:], res2)


# speedup vs baseline: 1.0743x; 1.0021x over previous
import functools

import jax
import jax.numpy as jnp
from jax import lax
from jax.experimental import pallas as pl
from jax.experimental.pallas import tpu as pltpu
from jax.experimental.pallas import tpu_sc as plsc

NC, NS, L = 2, 16, 16
NW = NC * NS
ROW1 = 144
ROW2 = 48
CH = 80


def _iota16():
    return lax.iota(jnp.int32, 16)


def _lane_gather(v, idx):
    return lax.gather(
        v, idx[:, None],
        dimension_numbers=lax.GatherDimensionNumbers(
            offset_dims=(), collapsed_slice_dims=(0,), start_index_map=(0,)),
        slice_sizes=(1,), mode=lax.GatherScatterMode.PROMISE_IN_BOUNDS)


def _splat(v, j):
    return _lane_gather(v, jnp.full((16,), j, dtype=jnp.int32))


ACC1 = 144
ACC2 = 48


def _make_edge_pass(SRCW, DW, ACCW, CH, compute_chunk, async_scatter=True):
    def run(srctab, dsttab, srcidx3, dstidx3, zeros):
        N = srctab.shape[0]
        steps = srcidx3.shape[1]
        rps = N // NS
        mesh = plsc.VectorSubcoreMesh(core_axis_name="c", subcore_axis_name="s",
                                      num_cores=NC, num_subcores=NS)

        @functools.partial(
            pl.kernel,
            out_type=jax.ShapeDtypeStruct((NC * N, ACCW), jnp.float32),
            mesh=mesh,
            scratch_types=[
                pltpu.VMEM((4, CH), jnp.int32),
                pltpu.VMEM((4, CH), jnp.int32),
                pltpu.VMEM((2, CH, SRCW), jnp.float32),
                pltpu.VMEM((2, CH, DW), jnp.float32),
                pltpu.VMEM((2 if async_scatter else 1, CH, ACCW), jnp.float32),
                pltpu.VMEM_SHARED((N, ACCW), jnp.float32),
                pltpu.SemaphoreType.DMA((4,)),
                pltpu.SemaphoreType.DMA((2,)),
                pltpu.SemaphoreType.DMA((2,)),
                pltpu.SemaphoreType.DMA((2,)),
            ],
            compiler_params=pltpu.CompilerParams(
                use_tc_tiling_on_sc=False, needs_layout_passes=False),
        )
        def k(srctab_hbm, dsttab_hbm, sidx_hbm, didx_hbm, zeros_hbm, out_hbm,
              sidx, didx, srows, drows, contrib, accum, isem, gs, gd, ssem):
            c = lax.axis_index("c")
            s = lax.axis_index("s")
            wid = s * NC + c
            r0 = s * rps

            pltpu.sync_copy(zeros_hbm, accum.at[pl.ds(r0, rps)])

            def issue_idx(t):
                slot = jnp.bitwise_and(t, 3)
                pltpu.async_copy(sidx_hbm.at[wid, t], sidx.at[slot],
                                 isem.at[slot])
                pltpu.async_copy(didx_hbm.at[wid, t], didx.at[slot],
                                 isem.at[slot])

            def wait_idx(t):
                slot = jnp.bitwise_and(t, 3)
                pltpu.make_async_copy(sidx_hbm.at[wid, t], sidx.at[slot],
                                      isem.at[slot]).wait()
                pltpu.make_async_copy(didx_hbm.at[wid, t], didx.at[slot],
                                      isem.at[slot]).wait()

            def issue_g(t, p):
                slot = jnp.bitwise_and(t, 3)
                pltpu.async_copy(srctab_hbm.at[sidx.at[slot]], srows.at[p],
                                 gs.at[p])
                pltpu.async_copy(dsttab_hbm.at[didx.at[slot]], drows.at[p],
                                 gd.at[p])

            def wait_g(t, p):
                slot = jnp.bitwise_and(t, 3)
                pltpu.make_async_copy(srctab_hbm.at[sidx.at[slot]],
                                      srows.at[p], gs.at[p]).wait()
                pltpu.make_async_copy(dsttab_hbm.at[didx.at[slot]],
                                      drows.at[p], gd.at[p]).wait()

            plsc.subcore_barrier()
            issue_idx(0)
            issue_idx(1)
            wait_idx(0)
            issue_g(0, 0)

            def drain_scatter(t):
                slot = jnp.bitwise_and(t, 3)
                p = jnp.bitwise_and(t, 1)
                pltpu.make_async_copy(contrib.at[p],
                                      accum.at[didx.at[slot]],
                                      ssem.at[p]).wait()

            def step(t, _):
                p = jnp.bitwise_and(t, 1)

                @pl.when(t + 2 < steps)
                def _():
                    issue_idx(t + 2)

                @pl.when(t + 1 < steps)
                def _():
                    wait_idx(t + 1)
                    issue_g(t + 1, 1 - p)
                wait_g(t, p)
                slot = jnp.bitwise_and(t, 3)
                if async_scatter:
                    @pl.when(t >= 2)
                    def _():
                        drain_scatter(t - 2)

                    @pl.when(p == 0)
                    def _():
                        compute_chunk(srows.at[0], drows.at[0], contrib.at[0])

                    @pl.when(p == 1)
                    def _():
                        compute_chunk(srows.at[1], drows.at[1], contrib.at[1])
                    pltpu.async_copy(contrib.at[p], accum.at[didx.at[slot]],
                                     ssem.at[p], add=True)
                else:
                    @pl.when(p == 0)
                    def _():
                        compute_chunk(srows.at[0], drows.at[0], contrib.at[0])

                    @pl.when(p == 1)
                    def _():
                        compute_chunk(srows.at[1], drows.at[1], contrib.at[0])
                    pltpu.sync_copy(contrib.at[0], accum.at[didx.at[slot]],
                                    add=True)
                return 0
            lax.fori_loop(0, steps, step, 0)
            if async_scatter:
                drain_scatter(steps - 2)
                drain_scatter(steps - 1)
            plsc.subcore_barrier()

            pltpu.sync_copy(accum.at[pl.ds(r0, rps)],
                            out_hbm.at[pl.ds(c * N + r0, rps)])

        return k(srctab, dsttab, srcidx3, dstidx3, zeros)

    return run


def _compute1(CH):
    def f(srows, drows, contrib):
        cvec = jnp.where(_iota16() < 8, 1.0, 0.2).astype(jnp.float32)
        swap = jnp.bitwise_and(_iota16() + 8, 15)

        @plsc.parallel_loop(0, CH, 1, unroll=8)
        def edge(i):
            el16 = srows[i, pl.ds(0, 16)]
            ab = drows[i, :]
            q = el16 * cvec + ab
            w16 = jnp.exp(jnp.maximum(q, _lane_gather(q, swap)))
            contrib[i, pl.ds(0, 16)] = w16
            for hh in range(8):
                hv = srows[i, pl.ds(16 + 16 * hh, 16)]
                contrib[i, pl.ds(16 + 16 * hh, 16)] = hv * _splat(w16, hh)
    return f


def _compute2(CH):
    def f(srows, drows, contrib):
        iota = _iota16()
        zeros_i = jnp.zeros((16,), jnp.int32)
        ones_i = jnp.ones((16,), jnp.int32)

        @plsc.parallel_loop(0, CH // 16, 1)
        def group(g):
            evec = g * 16 + iota
            el16 = plsc.load_gather(srows, [evec, zeros_i])
            a16 = plsc.load_gather(drows, [evec, zeros_i])
            b16 = plsc.load_gather(drows, [evec, ones_i])
            w16 = jnp.exp(jnp.maximum(el16 + a16, 0.2 * el16 + b16))
            for j in range(16):
                e = g * 16 + j
                wsp = _splat(w16, j)
                row0 = srows[e, pl.ds(0, 16)]
                row0 = jnp.where(iota == 0, 1.0, row0)
                contrib[e, pl.ds(0, 16)] = row0 * wsp
                contrib[e, pl.ds(16, 16)] = srows[e, pl.ds(16, 16)] * wsp
                contrib[e, pl.ds(32, 16)] = srows[e, pl.ds(32, 16)] * wsp
    return f


CH1, CH2 = 80, 80
_edge_pass1_run = _make_edge_pass(ROW1, 16, ACC1, CH1, _compute1(CH1),
                                  async_scatter=False)
_edge_pass2_run = _make_edge_pass(ROW2, 8, ACC2, CH2, _compute2(CH2))


def _tc_prep1(x, W1, ALR, interpret=False):
    N = x.shape[0]
    BN = 1000

    def body(x_ref, w_ref, alr_ref, src_ref, elr_ref, mx_ref):
        i = pl.program_id(0)
        xb = x_ref[...]
        hb = jnp.dot(xb, w_ref[...], preferred_element_type=jnp.float32)
        elr = jnp.dot(hb, alr_ref[...], preferred_element_type=jnp.float32)
        el = elr[:, 0:8]
        src_ref[...] = jnp.concatenate([el, el, hb], axis=1)
        elr_ref[...] = elr
        bmax = jnp.max(el, axis=0, keepdims=True)

        @pl.when(i == 0)
        def _():
            mx_ref[...] = bmax

        @pl.when(i > 0)
        def _():
            mx_ref[...] = jnp.maximum(mx_ref[...], bmax)

    return pl.pallas_call(
        body,
        grid=(N // BN,),
        in_specs=[pl.BlockSpec((BN, 128), lambda i: (i, 0)),
                  pl.BlockSpec((128, 128), lambda i: (0, 0)),
                  pl.BlockSpec((128, 16), lambda i: (0, 0))],
        out_specs=[pl.BlockSpec((BN, ROW1), lambda i: (i, 0)),
                   pl.BlockSpec((BN, 16), lambda i: (i, 0)),
                   pl.BlockSpec((1, 8), lambda i: (0, 0))],
        out_shape=[jax.ShapeDtypeStruct((N, ROW1), jnp.float32),
                   jax.ShapeDtypeStruct((N, 16), jnp.float32),
                   jax.ShapeDtypeStruct((1, 8), jnp.float32)],
        interpret=interpret,
    )(x, W1, ALR)


def _tc_dsttab1(elr, maxel, interpret=False):
    N = elr.shape[0]
    BN = 1000

    def body(elr_ref, mx_ref, out_ref):
        er = elr_ref[...][:, 8:16]
        t = er + mx_ref[...]
        m = jnp.where(t > 0, t, 0.2 * t)
        out_ref[...] = jnp.concatenate([er - m, 0.2 * er - m], axis=1)

    return pl.pallas_call(
        body,
        grid=(N // BN,),
        in_specs=[pl.BlockSpec((BN, 16), lambda i: (i, 0)),
                  pl.BlockSpec((1, 8), lambda i: (0, 0))],
        out_specs=pl.BlockSpec((BN, 16), lambda i: (i, 0)),
        out_shape=jax.ShapeDtypeStruct((N, 16), jnp.float32),
        interpret=interpret,
    )(elr, maxel)


def _tc_prep2(p0, p1, b1, W2, al2, ar2, resW2, b2, interpret=False):
    N = p0.shape[0]
    BN = 1000

    def body(p0_ref, p1_ref, b1_ref, w2_ref, al2_ref, ar2_ref, rw_ref, b2_ref,
             src_ref, elr_ref, res_ref, mx_ref):
        i = pl.program_id(0)
        acc = p0_ref[...] + p1_ref[...]
        num = acc[:, 16:ACC1]
        parts = []
        for hh in range(8):
            dh = acc[:, hh:hh + 1]
            parts.append(num[:, 16 * hh:16 * hh + 16] / (dh + 1e-16))
        rst = jnp.concatenate(parts, axis=1) + b1_ref[...]
        h2 = jnp.where(rst > 0, rst, jnp.exp(rst) - 1.0)
        h2w = jnp.dot(h2, w2_ref[...], preferred_element_type=jnp.float32)
        el2 = jnp.dot(h2w, al2_ref[...], preferred_element_type=jnp.float32)
        er2 = jnp.dot(h2w, ar2_ref[...], preferred_element_type=jnp.float32)
        res = jnp.dot(h2, rw_ref[...], preferred_element_type=jnp.float32)
        res_ref[...] = res + b2_ref[...]
        src_ref[...] = jnp.concatenate(
            [el2, h2w, jnp.zeros((h2w.shape[0], 7), jnp.float32)], axis=1)
        elr_ref[...] = jnp.concatenate(
            [el2, er2, jnp.zeros((h2w.shape[0], 14), jnp.float32)], axis=1)
        bmax = jnp.max(el2, axis=0, keepdims=True)

        @pl.when(i == 0)
        def _():
            mx_ref[...] = bmax

        @pl.when(i > 0)
        def _():
            mx_ref[...] = jnp.maximum(mx_ref[...], bmax)

    return pl.pallas_call(
        body,
        grid=(N // BN,),
        in_specs=[pl.BlockSpec((BN, ACC1), lambda i: (i, 0)),
                  pl.BlockSpec((BN, ACC1), lambda i: (i, 0)),
                  pl.BlockSpec((1, 128), lambda i: (0, 0)),
                  pl.BlockSpec((128, 40), lambda i: (0, 0)),
                  pl.BlockSpec((40, 1), lambda i: (0, 0)),
                  pl.BlockSpec((40, 1), lambda i: (0, 0)),
                  pl.BlockSpec((128, 40), lambda i: (0, 0)),
                  pl.BlockSpec((1, 40), lambda i: (0, 0))],
        out_specs=[pl.BlockSpec((BN, ROW2), lambda i: (i, 0)),
                   pl.BlockSpec((BN, 16), lambda i: (i, 0)),
                   pl.BlockSpec((BN, 40), lambda i: (i, 0)),
                   pl.BlockSpec((1, 1), lambda i: (0, 0))],
        out_shape=[jax.ShapeDtypeStruct((N, ROW2), jnp.float32),
                   jax.ShapeDtypeStruct((N, 16), jnp.float32),
                   jax.ShapeDtypeStruct((N, 40), jnp.float32),
                   jax.ShapeDtypeStruct((1, 1), jnp.float32)],
        interpret=interpret,
    )(p0, p1, b1, W2, al2, ar2, resW2, b2)


def _tc_dsttab2(elr2, maxel2, interpret=False):
    N = elr2.shape[0]
    BN = 1000

    def body(elr_ref, mx_ref, out_ref):
        e = elr_ref[...]
        er2 = e[:, 1:2]
        t = er2 + mx_ref[...]
        m = jnp.where(t > 0, t, 0.2 * t)
        out_ref[...] = jnp.concatenate(
            [er2 - m, 0.2 * er2 - m,
             jnp.zeros((e.shape[0], 6), jnp.float32)], axis=1)

    return pl.pallas_call(
        body,
        grid=(N // BN,),
        in_specs=[pl.BlockSpec((BN, 16), lambda i: (i, 0)),
                  pl.BlockSpec((1, 1), lambda i: (0, 0))],
        out_specs=pl.BlockSpec((BN, 8), lambda i: (i, 0)),
        out_shape=jax.ShapeDtypeStruct((N, 8), jnp.float32),
        interpret=interpret,
    )(elr2, maxel2)


def _tc_final(q0, q1, res, interpret=False):
    N = q0.shape[0]
    BN = 1000

    def body(q0_ref, q1_ref, res_ref, out_ref):
        acc = q0_ref[...] + q1_ref[...]
        out_ref[...] = acc[:, 1:41] / (acc[:, 0:1] + 1e-16) + res_ref[...]

    return pl.pallas_call(
        body,
        grid=(N // BN,),
        in_specs=[pl.BlockSpec((BN, ROW2), lambda i: (i, 0)),
                  pl.BlockSpec((BN, ROW2), lambda i: (i, 0)),
                  pl.BlockSpec((BN, 40), lambda i: (i, 0))],
        out_specs=pl.BlockSpec((BN, 40), lambda i: (i, 0)),
        out_shape=jax.ShapeDtypeStruct((N, 40), jnp.float32),
        interpret=interpret,
    )(q0, q1, res)


def kernel(x, edge_index, W1, al1, ar1, b1, W2, al2, ar2, b2, resW2):
    N = x.shape[0]
    src = edge_index[0]
    dst = edge_index[1]

    alr_l = (jnp.eye(8, dtype=jnp.float32)[:, None, :] * al1[:, :, None]
             ).reshape(128, 8)
    alr_r = (jnp.eye(8, dtype=jnp.float32)[:, None, :] * ar1[:, :, None]
             ).reshape(128, 8)
    ALR = jnp.concatenate([alr_l, alr_r], axis=1)

    E = src.shape[0]
    EW = E // NW
    src1 = src.reshape(NW, EW // CH1, CH1)
    dst1 = dst.reshape(NW, EW // CH1, CH1)
    src2 = src.reshape(NW, EW // CH2, CH2)
    dst2 = dst.reshape(NW, EW // CH2, CH2)
    zeros1 = jnp.zeros((N // NS, ACC1), jnp.float32)
    zeros2 = jnp.zeros((N // NS, ACC2), jnp.float32)

    srctab1, elr1, maxel1 = _tc_prep1(x, W1, ALR)
    dsttab1 = _tc_dsttab1(elr1, maxel1)
    part1 = _edge_pass1_run(srctab1, dsttab1, src1, dst1, zeros1)
    p0, p1 = part1[:N], part1[N:]

    srctab2, elr2, res2, maxel2 = _tc_prep2(
        p0, p1, b1.reshape(1, 128), W2, al2.reshape(40, 1), ar2.reshape(40, 1),
        resW2, b2.reshape(1, 40))
    dsttab2 = _tc_dsttab2(elr2, maxel2)
    part2 = _edge_pass2_run(srctab2, dsttab2, src2, dst2, zeros2)

    return _tc_final(part2[:N], part2[N:], res2)


# speedup vs baseline: 1.0789x; 1.0043x over previous
import functools

import jax
import jax.numpy as jnp
from jax import lax
from jax.experimental import pallas as pl
from jax.experimental.pallas import tpu as pltpu
from jax.experimental.pallas import tpu_sc as plsc

NC, NS, L = 2, 16, 16
NW = NC * NS
ROW1 = 144
ROW2 = 48
CH = 80


def _iota16():
    return lax.iota(jnp.int32, 16)


def _lane_gather(v, idx):
    return lax.gather(
        v, idx[:, None],
        dimension_numbers=lax.GatherDimensionNumbers(
            offset_dims=(), collapsed_slice_dims=(0,), start_index_map=(0,)),
        slice_sizes=(1,), mode=lax.GatherScatterMode.PROMISE_IN_BOUNDS)


def _splat(v, j):
    return _lane_gather(v, jnp.full((16,), j, dtype=jnp.int32))


ACC1 = 144
ACC2 = 48


def _make_edge_pass(SRCW, DW, ACCW, CH, compute_chunk, async_scatter=True):
    def run(srctab, dsttab, srcidx3, dstidx3, zeros):
        N = srctab.shape[0]
        steps = srcidx3.shape[1]
        rps = N // NS
        mesh = plsc.VectorSubcoreMesh(core_axis_name="c", subcore_axis_name="s",
                                      num_cores=NC, num_subcores=NS)

        @functools.partial(
            pl.kernel,
            out_type=jax.ShapeDtypeStruct((NC * N, ACCW), jnp.float32),
            mesh=mesh,
            scratch_types=[
                pltpu.VMEM((4, CH), jnp.int32),
                pltpu.VMEM((4, CH), jnp.int32),
                pltpu.VMEM((2, CH, SRCW), jnp.float32),
                pltpu.VMEM((2, CH, DW), jnp.float32),
                pltpu.VMEM((2 if async_scatter else 1, CH, ACCW), jnp.float32),
                pltpu.VMEM_SHARED((N, ACCW), jnp.float32),
                pltpu.SemaphoreType.DMA((4,)),
                pltpu.SemaphoreType.DMA((2,)),
                pltpu.SemaphoreType.DMA((2,)),
                pltpu.SemaphoreType.DMA((2,)),
            ],
            compiler_params=pltpu.CompilerParams(
                use_tc_tiling_on_sc=False, needs_layout_passes=False),
        )
        def k(srctab_hbm, dsttab_hbm, sidx_hbm, didx_hbm, zeros_hbm, out_hbm,
              sidx, didx, srows, drows, contrib, accum, isem, gs, gd, ssem):
            c = lax.axis_index("c")
            s = lax.axis_index("s")
            wid = s * NC + c
            r0 = s * rps

            pltpu.sync_copy(zeros_hbm, accum.at[pl.ds(r0, rps)])

            def issue_idx(t):
                slot = jnp.bitwise_and(t, 3)
                pltpu.async_copy(sidx_hbm.at[wid, t], sidx.at[slot],
                                 isem.at[slot])
                pltpu.async_copy(didx_hbm.at[wid, t], didx.at[slot],
                                 isem.at[slot])

            def wait_idx(t):
                slot = jnp.bitwise_and(t, 3)
                pltpu.make_async_copy(sidx_hbm.at[wid, t], sidx.at[slot],
                                      isem.at[slot]).wait()
                pltpu.make_async_copy(didx_hbm.at[wid, t], didx.at[slot],
                                      isem.at[slot]).wait()

            def issue_g(t, p):
                slot = jnp.bitwise_and(t, 3)
                pltpu.async_copy(srctab_hbm.at[sidx.at[slot]], srows.at[p],
                                 gs.at[p])
                pltpu.async_copy(dsttab_hbm.at[didx.at[slot]], drows.at[p],
                                 gd.at[p])

            def wait_g(t, p):
                slot = jnp.bitwise_and(t, 3)
                pltpu.make_async_copy(srctab_hbm.at[sidx.at[slot]],
                                      srows.at[p], gs.at[p]).wait()
                pltpu.make_async_copy(dsttab_hbm.at[didx.at[slot]],
                                      drows.at[p], gd.at[p]).wait()

            plsc.subcore_barrier()
            issue_idx(0)
            issue_idx(1)
            wait_idx(0)
            issue_g(0, 0)

            def drain_scatter(t):
                slot = jnp.bitwise_and(t, 3)
                p = jnp.bitwise_and(t, 1)
                pltpu.make_async_copy(contrib.at[p],
                                      accum.at[didx.at[slot]],
                                      ssem.at[p]).wait()

            def step(t, _):
                p = jnp.bitwise_and(t, 1)

                @pl.when(t + 2 < steps)
                def _():
                    issue_idx(t + 2)

                @pl.when(t + 1 < steps)
                def _():
                    wait_idx(t + 1)
                    issue_g(t + 1, 1 - p)
                wait_g(t, p)
                slot = jnp.bitwise_and(t, 3)
                if async_scatter:
                    @pl.when(t >= 2)
                    def _():
                        drain_scatter(t - 2)
                    compute_chunk(srows, drows, contrib, p, p)
                    pltpu.async_copy(contrib.at[p], accum.at[didx.at[slot]],
                                     ssem.at[p], add=True)
                else:
                    compute_chunk(srows, drows, contrib, p, 0)
                    pltpu.sync_copy(contrib.at[0], accum.at[didx.at[slot]],
                                    add=True)
                return 0
            lax.fori_loop(0, steps, step, 0)
            if async_scatter:
                drain_scatter(steps - 2)
                drain_scatter(steps - 1)
            plsc.subcore_barrier()

            pltpu.sync_copy(accum.at[pl.ds(r0, rps)],
                            out_hbm.at[pl.ds(c * N + r0, rps)])

        return k(srctab, dsttab, srcidx3, dstidx3, zeros)

    return run


def _compute1(CH):
    def f(srows, drows, contrib, p, cq):
        cvec = jnp.where(_iota16() < 8, 1.0, 0.2).astype(jnp.float32)
        swap = jnp.bitwise_and(_iota16() + 8, 15)

        @plsc.parallel_loop(0, CH, 1, unroll=4)
        def edge(i):
            el16 = srows[p, i, pl.ds(0, 16)]
            ab = drows[p, i, :]
            q = el16 * cvec + ab
            w16 = jnp.exp(jnp.maximum(q, _lane_gather(q, swap)))
            contrib[cq, i, pl.ds(0, 16)] = w16
            for hh in range(8):
                hv = srows[p, i, pl.ds(16 + 16 * hh, 16)]
                contrib[cq, i, pl.ds(16 + 16 * hh, 16)] = hv * _splat(w16, hh)
    return f


def _compute2(CH):
    def f(srows, drows, contrib, p, cq):
        iota = _iota16()
        zeros_i = jnp.zeros((16,), jnp.int32)
        ones_i = jnp.ones((16,), jnp.int32)
        pfull = jnp.full((16,), p, dtype=jnp.int32)

        @plsc.parallel_loop(0, CH // 16, 1)
        def group(g):
            evec = g * 16 + iota
            el16 = plsc.load_gather(srows, [pfull, evec, zeros_i])
            a16 = plsc.load_gather(drows, [pfull, evec, zeros_i])
            b16 = plsc.load_gather(drows, [pfull, evec, ones_i])
            w16 = jnp.exp(jnp.maximum(el16 + a16, 0.2 * el16 + b16))
            for j in range(16):
                e = g * 16 + j
                wsp = _splat(w16, j)
                row0 = srows[p, e, pl.ds(0, 16)]
                row0 = jnp.where(iota == 0, 1.0, row0)
                contrib[cq, e, pl.ds(0, 16)] = row0 * wsp
                contrib[cq, e, pl.ds(16, 16)] = srows[p, e, pl.ds(16, 16)] * wsp
                contrib[cq, e, pl.ds(32, 16)] = srows[p, e, pl.ds(32, 16)] * wsp
    return f


CH1, CH2 = 80, 80
_edge_pass1_run = _make_edge_pass(ROW1, 16, ACC1, CH1, _compute1(CH1),
                                  async_scatter=False)
_edge_pass2_run = _make_edge_pass(ROW2, 8, ACC2, CH2, _compute2(CH2))


def _tc_prep1(x, W1, ALR, interpret=False):
    N = x.shape[0]
    BN = 1000

    def body(x_ref, w_ref, alr_ref, src_ref, elr_ref, mx_ref):
        i = pl.program_id(0)
        xb = x_ref[...]
        hb = jnp.dot(xb, w_ref[...], preferred_element_type=jnp.float32)
        elr = jnp.dot(hb, alr_ref[...], preferred_element_type=jnp.float32)
        el = elr[:, 0:8]
        src_ref[...] = jnp.concatenate([el, el, hb], axis=1)
        elr_ref[...] = elr
        bmax = jnp.max(el, axis=0, keepdims=True)

        @pl.when(i == 0)
        def _():
            mx_ref[...] = bmax

        @pl.when(i > 0)
        def _():
            mx_ref[...] = jnp.maximum(mx_ref[...], bmax)

    return pl.pallas_call(
        body,
        grid=(N // BN,),
        in_specs=[pl.BlockSpec((BN, 128), lambda i: (i, 0)),
                  pl.BlockSpec((128, 128), lambda i: (0, 0)),
                  pl.BlockSpec((128, 16), lambda i: (0, 0))],
        out_specs=[pl.BlockSpec((BN, ROW1), lambda i: (i, 0)),
                   pl.BlockSpec((BN, 16), lambda i: (i, 0)),
                   pl.BlockSpec((1, 8), lambda i: (0, 0))],
        out_shape=[jax.ShapeDtypeStruct((N, ROW1), jnp.float32),
                   jax.ShapeDtypeStruct((N, 16), jnp.float32),
                   jax.ShapeDtypeStruct((1, 8), jnp.float32)],
        interpret=interpret,
    )(x, W1, ALR)


def _tc_dsttab1(elr, maxel, interpret=False):
    N = elr.shape[0]
    BN = 1000

    def body(elr_ref, mx_ref, out_ref):
        er = elr_ref[...][:, 8:16]
        t = er + mx_ref[...]
        m = jnp.where(t > 0, t, 0.2 * t)
        out_ref[...] = jnp.concatenate([er - m, 0.2 * er - m], axis=1)

    return pl.pallas_call(
        body,
        grid=(N // BN,),
        in_specs=[pl.BlockSpec((BN, 16), lambda i: (i, 0)),
                  pl.BlockSpec((1, 8), lambda i: (0, 0))],
        out_specs=pl.BlockSpec((BN, 16), lambda i: (i, 0)),
        out_shape=jax.ShapeDtypeStruct((N, 16), jnp.float32),
        interpret=interpret,
    )(elr, maxel)


def _tc_prep2(p0, p1, b1, W2, al2, ar2, resW2, b2, interpret=False):
    N = p0.shape[0]
    BN = 1000

    def body(p0_ref, p1_ref, b1_ref, w2_ref, al2_ref, ar2_ref, rw_ref, b2_ref,
             src_ref, elr_ref, res_ref, mx_ref):
        i = pl.program_id(0)
        acc = p0_ref[...] + p1_ref[...]
        num = acc[:, 16:ACC1]
        parts = []
        for hh in range(8):
            dh = acc[:, hh:hh + 1]
            parts.append(num[:, 16 * hh:16 * hh + 16] / (dh + 1e-16))
        rst = jnp.concatenate(parts, axis=1) + b1_ref[...]
        h2 = jnp.where(rst > 0, rst, jnp.exp(rst) - 1.0)
        h2w = jnp.dot(h2, w2_ref[...], preferred_element_type=jnp.float32)
        el2 = jnp.dot(h2w, al2_ref[...], preferred_element_type=jnp.float32)
        er2 = jnp.dot(h2w, ar2_ref[...], preferred_element_type=jnp.float32)
        res = jnp.dot(h2, rw_ref[...], preferred_element_type=jnp.float32)
        res_ref[...] = res + b2_ref[...]
        src_ref[...] = jnp.concatenate(
            [el2, h2w, jnp.zeros((h2w.shape[0], 7), jnp.float32)], axis=1)
        elr_ref[...] = jnp.concatenate(
            [el2, er2, jnp.zeros((h2w.shape[0], 14), jnp.float32)], axis=1)
        bmax = jnp.max(el2, axis=0, keepdims=True)

        @pl.when(i == 0)
        def _():
            mx_ref[...] = bmax

        @pl.when(i > 0)
        def _():
            mx_ref[...] = jnp.maximum(mx_ref[...], bmax)

    return pl.pallas_call(
        body,
        grid=(N // BN,),
        in_specs=[pl.BlockSpec((BN, ACC1), lambda i: (i, 0)),
                  pl.BlockSpec((BN, ACC1), lambda i: (i, 0)),
                  pl.BlockSpec((1, 128), lambda i: (0, 0)),
                  pl.BlockSpec((128, 40), lambda i: (0, 0)),
                  pl.BlockSpec((40, 1), lambda i: (0, 0)),
                  pl.BlockSpec((40, 1), lambda i: (0, 0)),
                  pl.BlockSpec((128, 40), lambda i: (0, 0)),
                  pl.BlockSpec((1, 40), lambda i: (0, 0))],
        out_specs=[pl.BlockSpec((BN, ROW2), lambda i: (i, 0)),
                   pl.BlockSpec((BN, 16), lambda i: (i, 0)),
                   pl.BlockSpec((BN, 40), lambda i: (i, 0)),
                   pl.BlockSpec((1, 1), lambda i: (0, 0))],
        out_shape=[jax.ShapeDtypeStruct((N, ROW2), jnp.float32),
                   jax.ShapeDtypeStruct((N, 16), jnp.float32),
                   jax.ShapeDtypeStruct((N, 40), jnp.float32),
                   jax.ShapeDtypeStruct((1, 1), jnp.float32)],
        interpret=interpret,
    )(p0, p1, b1, W2, al2, ar2, resW2, b2)


def _tc_dsttab2(elr2, maxel2, interpret=False):
    N = elr2.shape[0]
    BN = 1000

    def body(elr_ref, mx_ref, out_ref):
        e = elr_ref[...]
        er2 = e[:, 1:2]
        t = er2 + mx_ref[...]
        m = jnp.where(t > 0, t, 0.2 * t)
        out_ref[...] = jnp.concatenate(
            [er2 - m, 0.2 * er2 - m,
             jnp.zeros((e.shape[0], 6), jnp.float32)], axis=1)

    return pl.pallas_call(
        body,
        grid=(N // BN,),
        in_specs=[pl.BlockSpec((BN, 16), lambda i: (i, 0)),
                  pl.BlockSpec((1, 1), lambda i: (0, 0))],
        out_specs=pl.BlockSpec((BN, 8), lambda i: (i, 0)),
        out_shape=jax.ShapeDtypeStruct((N, 8), jnp.float32),
        interpret=interpret,
    )(elr2, maxel2)


def _tc_final(q0, q1, res, interpret=False):
    N = q0.shape[0]
    BN = 1000

    def body(q0_ref, q1_ref, res_ref, out_ref):
        acc = q0_ref[...] + q1_ref[...]
        out_ref[...] = acc[:, 1:41] / (acc[:, 0:1] + 1e-16) + res_ref[...]

    return pl.pallas_call(
        body,
        grid=(N // BN,),
        in_specs=[pl.BlockSpec((BN, ROW2), lambda i: (i, 0)),
                  pl.BlockSpec((BN, ROW2), lambda i: (i, 0)),
                  pl.BlockSpec((BN, 40), lambda i: (i, 0))],
        out_specs=pl.BlockSpec((BN, 40), lambda i: (i, 0)),
        out_shape=jax.ShapeDtypeStruct((N, 40), jnp.float32),
        interpret=interpret,
    )(q0, q1, res)


def kernel(x, edge_index, W1, al1, ar1, b1, W2, al2, ar2, b2, resW2):
    N = x.shape[0]
    src = edge_index[0]
    dst = edge_index[1]

    alr_l = (jnp.eye(8, dtype=jnp.float32)[:, None, :] * al1[:, :, None]
             ).reshape(128, 8)
    alr_r = (jnp.eye(8, dtype=jnp.float32)[:, None, :] * ar1[:, :, None]
             ).reshape(128, 8)
    ALR = jnp.concatenate([alr_l, alr_r], axis=1)

    E = src.shape[0]
    EW = E // NW
    src1 = src.reshape(NW, EW // CH1, CH1)
    dst1 = dst.reshape(NW, EW // CH1, CH1)
    src2 = src.reshape(NW, EW // CH2, CH2)
    dst2 = dst.reshape(NW, EW // CH2, CH2)
    zeros1 = jnp.zeros((N // NS, ACC1), jnp.float32)
    zeros2 = jnp.zeros((N // NS, ACC2), jnp.float32)

    srctab1, elr1, maxel1 = _tc_prep1(x, W1, ALR)
    dsttab1 = _tc_dsttab1(elr1, maxel1)
    part1 = _edge_pass1_run(srctab1, dsttab1, src1, dst1, zeros1)
    p0, p1 = part1[:N], part1[N:]

    srctab2, elr2, res2, maxel2 = _tc_prep2(
        p0, p1, b1.reshape(1, 128), W2, al2.reshape(40, 1), ar2.reshape(40, 1),
        resW2, b2.reshape(1, 40))
    dsttab2 = _tc_dsttab2(elr2, maxel2)
    part2 = _edge_pass2_run(srctab2, dsttab2, src2, dst2, zeros2)

    return _tc_final(part2[:N], part2[N:], res2)


# speedup vs baseline: 1.0815x; 1.0024x over previous
import functools

import jax
import jax.numpy as jnp
from jax import lax
from jax.experimental import pallas as pl
from jax.experimental.pallas import tpu as pltpu
from jax.experimental.pallas import tpu_sc as plsc

NC, NS, L = 2, 16, 16
NW = NC * NS
ROW1 = 144
ROW2 = 48
CH = 80


def _iota16():
    return lax.iota(jnp.int32, 16)


def _lane_gather(v, idx):
    return lax.gather(
        v, idx[:, None],
        dimension_numbers=lax.GatherDimensionNumbers(
            offset_dims=(), collapsed_slice_dims=(0,), start_index_map=(0,)),
        slice_sizes=(1,), mode=lax.GatherScatterMode.PROMISE_IN_BOUNDS)


def _splat(v, j):
    return _lane_gather(v, jnp.full((16,), j, dtype=jnp.int32))


ACC1 = 144
ACC2 = 48


def _make_edge_pass(SRCW, DW, ACCW, CH, compute_chunk, async_scatter=True):
    def run(srctab, dsttab, srcidx3, dstidx3, zeros):
        N = srctab.shape[0]
        steps = srcidx3.shape[1]
        rps = N // NS
        mesh = plsc.VectorSubcoreMesh(core_axis_name="c", subcore_axis_name="s",
                                      num_cores=NC, num_subcores=NS)

        @functools.partial(
            pl.kernel,
            out_type=jax.ShapeDtypeStruct((NC * N, ACCW), jnp.float32),
            mesh=mesh,
            scratch_types=[
                pltpu.VMEM((4, CH), jnp.int32),
                pltpu.VMEM((4, CH), jnp.int32),
                pltpu.VMEM((2, CH, SRCW), jnp.float32),
                pltpu.VMEM((2, CH, DW), jnp.float32),
                pltpu.VMEM((2 if async_scatter else 1, CH, ACCW), jnp.float32),
                pltpu.VMEM_SHARED((N, ACCW), jnp.float32),
                pltpu.SemaphoreType.DMA((4,)),
                pltpu.SemaphoreType.DMA((2,)),
                pltpu.SemaphoreType.DMA((2,)),
                pltpu.SemaphoreType.DMA((2,)),
            ],
            compiler_params=pltpu.CompilerParams(
                use_tc_tiling_on_sc=False, needs_layout_passes=False),
        )
        def k(srctab_hbm, dsttab_hbm, sidx_hbm, didx_hbm, zeros_hbm, out_hbm,
              sidx, didx, srows, drows, contrib, accum, isem, gs, gd, ssem):
            c = lax.axis_index("c")
            s = lax.axis_index("s")
            wid = s * NC + c
            r0 = s * rps

            pltpu.sync_copy(zeros_hbm, accum.at[pl.ds(r0, rps)])

            def issue_idx(t):
                slot = jnp.bitwise_and(t, 3)
                pltpu.async_copy(sidx_hbm.at[wid, t], sidx.at[slot],
                                 isem.at[slot])
                pltpu.async_copy(didx_hbm.at[wid, t], didx.at[slot],
                                 isem.at[slot])

            def wait_idx(t):
                slot = jnp.bitwise_and(t, 3)
                pltpu.make_async_copy(sidx_hbm.at[wid, t], sidx.at[slot],
                                      isem.at[slot]).wait()
                pltpu.make_async_copy(didx_hbm.at[wid, t], didx.at[slot],
                                      isem.at[slot]).wait()

            def issue_g(t, p):
                slot = jnp.bitwise_and(t, 3)
                pltpu.async_copy(srctab_hbm.at[sidx.at[slot]], srows.at[p],
                                 gs.at[p])
                pltpu.async_copy(dsttab_hbm.at[didx.at[slot]], drows.at[p],
                                 gd.at[p])

            def wait_g(t, p):
                slot = jnp.bitwise_and(t, 3)
                pltpu.make_async_copy(srctab_hbm.at[sidx.at[slot]],
                                      srows.at[p], gs.at[p]).wait()
                pltpu.make_async_copy(dsttab_hbm.at[didx.at[slot]],
                                      drows.at[p], gd.at[p]).wait()

            plsc.subcore_barrier()
            issue_idx(0)
            issue_idx(1)
            wait_idx(0)
            issue_g(0, 0)

            def drain_scatter(t):
                slot = jnp.bitwise_and(t, 3)
                p = jnp.bitwise_and(t, 1)
                pltpu.make_async_copy(contrib.at[p],
                                      accum.at[didx.at[slot]],
                                      ssem.at[p]).wait()

            def step(t, _):
                p = jnp.bitwise_and(t, 1)

                @pl.when(t + 2 < steps)
                def _():
                    issue_idx(t + 2)

                @pl.when(t + 1 < steps)
                def _():
                    wait_idx(t + 1)
                    issue_g(t + 1, 1 - p)
                wait_g(t, p)
                slot = jnp.bitwise_and(t, 3)
                if async_scatter:
                    @pl.when(t >= 2)
                    def _():
                        drain_scatter(t - 2)
                    compute_chunk(srows, drows, contrib, p, p)
                    pltpu.async_copy(contrib.at[p], accum.at[didx.at[slot]],
                                     ssem.at[p], add=True)
                else:
                    compute_chunk(srows, drows, contrib, p, 0)
                    pltpu.sync_copy(contrib.at[0], accum.at[didx.at[slot]],
                                    add=True)
                return 0
            lax.fori_loop(0, steps, step, 0)
            if async_scatter:
                drain_scatter(steps - 2)
                drain_scatter(steps - 1)
            plsc.subcore_barrier()

            pltpu.sync_copy(accum.at[pl.ds(r0, rps)],
                            out_hbm.at[pl.ds(c * N + r0, rps)])

        return k(srctab, dsttab, srcidx3, dstidx3, zeros)

    return run


def _compute1(CH):
    def f(srows, drows, contrib, p, cq):
        cvec = jnp.where(_iota16() < 8, 1.0, 0.2).astype(jnp.float32)
        swap = jnp.bitwise_and(_iota16() + 8, 15)

        @plsc.parallel_loop(0, CH, 1, unroll=4)
        def edge(i):
            el16 = srows[p, i, pl.ds(0, 16)]
            ab = drows[p, i, :]
            q = el16 * cvec + ab
            w16 = jnp.exp(jnp.maximum(q, _lane_gather(q, swap)))
            contrib[cq, i, pl.ds(0, 16)] = w16
            for hh in range(8):
                hv = srows[p, i, pl.ds(16 + 16 * hh, 16)]
                contrib[cq, i, pl.ds(16 + 16 * hh, 16)] = hv * _splat(w16, hh)
    return f


def _compute2(CH):
    def f(srows, drows, contrib, p, cq):
        iota = _iota16()
        zeros_i = jnp.zeros((16,), jnp.int32)
        ones_i = jnp.ones((16,), jnp.int32)
        pfull = jnp.full((16,), p, dtype=jnp.int32)

        @plsc.parallel_loop(0, CH // 16, 1, unroll=2)
        def group(g):
            evec = g * 16 + iota
            el16 = plsc.load_gather(srows, [pfull, evec, zeros_i])
            a16 = plsc.load_gather(drows, [pfull, evec, zeros_i])
            b16 = plsc.load_gather(drows, [pfull, evec, ones_i])
            w16 = jnp.exp(jnp.maximum(el16 + a16, 0.2 * el16 + b16))
            for j in range(16):
                e = g * 16 + j
                wsp = _splat(w16, j)
                row0 = srows[p, e, pl.ds(0, 16)]
                row0 = jnp.where(iota == 0, 1.0, row0)
                contrib[cq, e, pl.ds(0, 16)] = row0 * wsp
                contrib[cq, e, pl.ds(16, 16)] = srows[p, e, pl.ds(16, 16)] * wsp
                contrib[cq, e, pl.ds(32, 16)] = srows[p, e, pl.ds(32, 16)] * wsp
    return f


CH1, CH2 = 80, 80
_edge_pass1_run = _make_edge_pass(ROW1, 16, ACC1, CH1, _compute1(CH1),
                                  async_scatter=False)
_edge_pass2_run = _make_edge_pass(ROW2, 8, ACC2, CH2, _compute2(CH2))


def _tc_prep1(x, W1, ALR, interpret=False):
    N = x.shape[0]
    BN = 1000

    def body(x_ref, w_ref, alr_ref, src_ref, elr_ref, mx_ref):
        i = pl.program_id(0)
        xb = x_ref[...]
        hb = jnp.dot(xb, w_ref[...], preferred_element_type=jnp.float32)
        elr = jnp.dot(hb, alr_ref[...], preferred_element_type=jnp.float32)
        el = elr[:, 0:8]
        src_ref[...] = jnp.concatenate([el, el, hb], axis=1)
        elr_ref[...] = elr
        bmax = jnp.max(el, axis=0, keepdims=True)

        @pl.when(i == 0)
        def _():
            mx_ref[...] = bmax

        @pl.when(i > 0)
        def _():
            mx_ref[...] = jnp.maximum(mx_ref[...], bmax)

    return pl.pallas_call(
        body,
        grid=(N // BN,),
        in_specs=[pl.BlockSpec((BN, 128), lambda i: (i, 0)),
                  pl.BlockSpec((128, 128), lambda i: (0, 0)),
                  pl.BlockSpec((128, 16), lambda i: (0, 0))],
        out_specs=[pl.BlockSpec((BN, ROW1), lambda i: (i, 0)),
                   pl.BlockSpec((BN, 16), lambda i: (i, 0)),
                   pl.BlockSpec((1, 8), lambda i: (0, 0))],
        out_shape=[jax.ShapeDtypeStruct((N, ROW1), jnp.float32),
                   jax.ShapeDtypeStruct((N, 16), jnp.float32),
                   jax.ShapeDtypeStruct((1, 8), jnp.float32)],
        interpret=interpret,
    )(x, W1, ALR)


def _tc_dsttab1(elr, maxel, interpret=False):
    N = elr.shape[0]
    BN = 1000

    def body(elr_ref, mx_ref, out_ref):
        er = elr_ref[...][:, 8:16]
        t = er + mx_ref[...]
        m = jnp.where(t > 0, t, 0.2 * t)
        out_ref[...] = jnp.concatenate([er - m, 0.2 * er - m], axis=1)

    return pl.pallas_call(
        body,
        grid=(N // BN,),
        in_specs=[pl.BlockSpec((BN, 16), lambda i: (i, 0)),
                  pl.BlockSpec((1, 8), lambda i: (0, 0))],
        out_specs=pl.BlockSpec((BN, 16), lambda i: (i, 0)),
        out_shape=jax.ShapeDtypeStruct((N, 16), jnp.float32),
        interpret=interpret,
    )(elr, maxel)


def _tc_prep2(p0, p1, b1, W2, al2, ar2, resW2, b2, interpret=False):
    N = p0.shape[0]
    BN = 1000

    def body(p0_ref, p1_ref, b1_ref, w2_ref, al2_ref, ar2_ref, rw_ref, b2_ref,
             src_ref, elr_ref, res_ref, mx_ref):
        i = pl.program_id(0)
        acc = p0_ref[...] + p1_ref[...]
        num = acc[:, 16:ACC1]
        parts = []
        for hh in range(8):
            dh = acc[:, hh:hh + 1]
            parts.append(num[:, 16 * hh:16 * hh + 16] / (dh + 1e-16))
        rst = jnp.concatenate(parts, axis=1) + b1_ref[...]
        h2 = jnp.where(rst > 0, rst, jnp.exp(rst) - 1.0)
        h2w = jnp.dot(h2, w2_ref[...], preferred_element_type=jnp.float32)
        el2 = jnp.dot(h2w, al2_ref[...], preferred_element_type=jnp.float32)
        er2 = jnp.dot(h2w, ar2_ref[...], preferred_element_type=jnp.float32)
        res = jnp.dot(h2, rw_ref[...], preferred_element_type=jnp.float32)
        res_ref[...] = res + b2_ref[...]
        src_ref[...] = jnp.concatenate(
            [el2, h2w, jnp.zeros((h2w.shape[0], 7), jnp.float32)], axis=1)
        elr_ref[...] = jnp.concatenate(
            [el2, er2, jnp.zeros((h2w.shape[0], 14), jnp.float32)], axis=1)
        bmax = jnp.max(el2, axis=0, keepdims=True)

        @pl.when(i == 0)
        def _():
            mx_ref[...] = bmax

        @pl.when(i > 0)
        def _():
            mx_ref[...] = jnp.maximum(mx_ref[...], bmax)

    return pl.pallas_call(
        body,
        grid=(N // BN,),
        in_specs=[pl.BlockSpec((BN, ACC1), lambda i: (i, 0)),
                  pl.BlockSpec((BN, ACC1), lambda i: (i, 0)),
                  pl.BlockSpec((1, 128), lambda i: (0, 0)),
                  pl.BlockSpec((128, 40), lambda i: (0, 0)),
                  pl.BlockSpec((40, 1), lambda i: (0, 0)),
                  pl.BlockSpec((40, 1), lambda i: (0, 0)),
                  pl.BlockSpec((128, 40), lambda i: (0, 0)),
                  pl.BlockSpec((1, 40), lambda i: (0, 0))],
        out_specs=[pl.BlockSpec((BN, ROW2), lambda i: (i, 0)),
                   pl.BlockSpec((BN, 16), lambda i: (i, 0)),
                   pl.BlockSpec((BN, 40), lambda i: (i, 0)),
                   pl.BlockSpec((1, 1), lambda i: (0, 0))],
        out_shape=[jax.ShapeDtypeStruct((N, ROW2), jnp.float32),
                   jax.ShapeDtypeStruct((N, 16), jnp.float32),
                   jax.ShapeDtypeStruct((N, 40), jnp.float32),
                   jax.ShapeDtypeStruct((1, 1), jnp.float32)],
        interpret=interpret,
    )(p0, p1, b1, W2, al2, ar2, resW2, b2)


def _tc_dsttab2(elr2, maxel2, interpret=False):
    N = elr2.shape[0]
    BN = 1000

    def body(elr_ref, mx_ref, out_ref):
        e = elr_ref[...]
        er2 = e[:, 1:2]
        t = er2 + mx_ref[...]
        m = jnp.where(t > 0, t, 0.2 * t)
        out_ref[...] = jnp.concatenate(
            [er2 - m, 0.2 * er2 - m,
             jnp.zeros((e.shape[0], 6), jnp.float32)], axis=1)

    return pl.pallas_call(
        body,
        grid=(N // BN,),
        in_specs=[pl.BlockSpec((BN, 16), lambda i: (i, 0)),
                  pl.BlockSpec((1, 1), lambda i: (0, 0))],
        out_specs=pl.BlockSpec((BN, 8), lambda i: (i, 0)),
        out_shape=jax.ShapeDtypeStruct((N, 8), jnp.float32),
        interpret=interpret,
    )(elr2, maxel2)


def _tc_final(q0, q1, res, interpret=False):
    N = q0.shape[0]
    BN = 1000

    def body(q0_ref, q1_ref, res_ref, out_ref):
        acc = q0_ref[...] + q1_ref[...]
        out_ref[...] = acc[:, 1:41] / (acc[:, 0:1] + 1e-16) + res_ref[...]

    return pl.pallas_call(
        body,
        grid=(N // BN,),
        in_specs=[pl.BlockSpec((BN, ROW2), lambda i: (i, 0)),
                  pl.BlockSpec((BN, ROW2), lambda i: (i, 0)),
                  pl.BlockSpec((BN, 40), lambda i: (i, 0))],
        out_specs=pl.BlockSpec((BN, 40), lambda i: (i, 0)),
        out_shape=jax.ShapeDtypeStruct((N, 40), jnp.float32),
        interpret=interpret,
    )(q0, q1, res)


def kernel(x, edge_index, W1, al1, ar1, b1, W2, al2, ar2, b2, resW2):
    N = x.shape[0]
    src = edge_index[0]
    dst = edge_index[1]

    alr_l = (jnp.eye(8, dtype=jnp.float32)[:, None, :] * al1[:, :, None]
             ).reshape(128, 8)
    alr_r = (jnp.eye(8, dtype=jnp.float32)[:, None, :] * ar1[:, :, None]
             ).reshape(128, 8)
    ALR = jnp.concatenate([alr_l, alr_r], axis=1)

    E = src.shape[0]
    EW = E // NW
    src1 = src.reshape(NW, EW // CH1, CH1)
    dst1 = dst.reshape(NW, EW // CH1, CH1)
    src2 = src.reshape(NW, EW // CH2, CH2)
    dst2 = dst.reshape(NW, EW // CH2, CH2)
    zeros1 = jnp.zeros((N // NS, ACC1), jnp.float32)
    zeros2 = jnp.zeros((N // NS, ACC2), jnp.float32)

    srctab1, elr1, maxel1 = _tc_prep1(x, W1, ALR)
    dsttab1 = _tc_dsttab1(elr1, maxel1)
    part1 = _edge_pass1_run(srctab1, dsttab1, src1, dst1, zeros1)
    p0, p1 = part1[:N], part1[N:]

    srctab2, elr2, res2, maxel2 = _tc_prep2(
        p0, p1, b1.reshape(1, 128), W2, al2.reshape(40, 1), ar2.reshape(40, 1),
        resW2, b2.reshape(1, 40))
    dsttab2 = _tc_dsttab2(elr2, maxel2)
    part2 = _edge_pass2_run(srctab2, dsttab2, src2, dst2, zeros2)

    return _tc_final(part2[:N], part2[N:], res2)
